# baseline (speedup 1.0000x reference)
import functools
import math

import jax
import jax.numpy as jnp
from jax import lax
from jax.experimental import pallas as pl
from jax.experimental.pallas import tpu as pltpu
from jax.experimental.pallas import tpu_sc as plsc

N = 10000
E = 320000
HID = 128
HEADS = 8
CH = HID // HEADS
TDIM = 32
NREL = 200
NEVAL = 131072

_EBLK = 8000
_C = 64
_ACCW = 144
_ESC = E // 2
_NCH = _ESC // _C
_NITER = 158


def _ale_body(t_ref, et_ref, ta_ref, rel_ref, o1_ref, o2_ref):
    half = TDIM // 2
    idx = lax.broadcasted_iota(jnp.int32, (1, half), 1).astype(jnp.float32)
    freqs = jnp.exp(-math.log(10000.0) * idx / half)
    t0 = t_ref[:, 0:1]
    t1 = t_ref[:, 1:2]
    ang0 = t0 * freqs
    ang1 = t1 * freqs

    def _sin(x):
        x2 = x * x
        return x * (1.0 + x2 * (-1.0 / 6.0 + x2 * (1.0 / 120.0 - x2 / 5040.0)))

    def _cos(x):
        x2 = x * x
        return 1.0 + x2 * (-0.5 + x2 * (1.0 / 24.0 + x2 * (-1.0 / 720.0
                                                           + x2 / 40320.0)))

    sincos = jnp.concatenate(
        [_sin(ang0), _cos(ang0), _sin(ang1), _cos(ang1)], axis=1)
    st = jnp.dot(sincos, ta_ref[...], preferred_element_type=jnp.float32)
    et = et_ref[0, 0, :]
    onehot = (et[:, None] == lax.broadcasted_iota(jnp.int32, (1, NREL), 1)
              ).astype(jnp.float32)
    st = st + jnp.dot(onehot, rel_ref[...], preferred_element_type=jnp.float32)
    o1_ref[...] = st[:, :8]
    o2_ref[...] = st[:, 8:]


def _ale_time(edge_t, edge_type, taec, relc):
    grid = (E // _EBLK,)
    et3 = edge_type.reshape(E // _EBLK, 1, _EBLK)
    return pl.pallas_call(
        _ale_body,
        grid=grid,
        in_specs=[
            pl.BlockSpec((_EBLK, 2), lambda i: (i, 0)),
            pl.BlockSpec((1, 1, _EBLK), lambda i: (i, 0, 0)),
            pl.BlockSpec((64, 16), lambda i: (0, 0)),
            pl.BlockSpec((NREL, 16), lambda i: (0, 0)),
        ],
        out_specs=[
            pl.BlockSpec((_EBLK, 8), lambda i: (i, 0)),
            pl.BlockSpec((_EBLK, 8), lambda i: (i, 0)),
        ],
        out_shape=[
            jax.ShapeDtypeStruct((E, 8), jnp.float32),
            jax.ShapeDtypeStruct((E, 8), jnp.float32),
        ],
    )(edge_t, et3, taec, relc)


def _edge_pass(src, dst, tbl, aldp, ale_flat):
    mesh = plsc.VectorSubcoreMesh(core_axis_name="c", subcore_axis_name="s")

    vm = pltpu.VMEM
    per_buf = [
        vm((_C,), jnp.int32), vm((_C,), jnp.int32),
        vm((_C, _ACCW), jnp.float32), vm((_C, 16), jnp.float32),
        vm((_C * 8 + 8,), jnp.float32), vm((_C, _ACCW), jnp.float32),
        pltpu.SemaphoreType.DMA, pltpu.SemaphoreType.DMA,
    ]

    @functools.partial(
        pl.kernel,
        out_type=jax.ShapeDtypeStruct((2, N, _ACCW), jnp.float32),
        mesh=mesh,
        scratch_types=per_buf + per_buf + [
            pltpu.VMEM_SHARED((N, _ACCW), jnp.float32),
        ],
        compiler_params=pltpu.CompilerParams(use_tc_tiling_on_sc=False),
    )
    def k(src_h, dst_h, tbl_h, ald_h, ale_h, out_h, *scr):
        acc = scr[-1]
        bufs = (scr[0:8], scr[8:16])
        c = lax.axis_index("c")
        s = lax.axis_index("s")
        zero16 = jnp.zeros((16,), jnp.float32)
        outb0 = bufs[0][5]

        def zrow(i, carry):
            for j in range(_ACCW // 16):
                outb0[i, pl.ds(j * 16, 16)] = zero16
            return carry
        lax.fori_loop(0, _C, zrow, 0)
        rows_per_tec = N // 16
        zbase = s * rows_per_tec
        for r in range(rows_per_tec // _C):
            pltpu.sync_copy(outb0, acc.at[pl.ds(zbase + r * _C, _C)])
        zrem = rows_per_tec % _C
        pltpu.sync_copy(outb0.at[pl.ds(0, zrem)],
                        acc.at[pl.ds(zbase + rows_per_tec - zrem, zrem)])
        plsc.subcore_barrier()

        def cid_of(kk):
            return s + 16 * kk

        def valid(kk):
            return cid_of(kk) < _NCH

        def issue_idx(kk, b):
            idx_s, idx_d, _, _, _, _, semi, _ = bufs[b]
            off = c * _ESC + cid_of(kk) * _C
            pltpu.async_copy(src_h.at[pl.ds(off, _C)], idx_s, semi)
            pltpu.async_copy(dst_h.at[pl.ds(off, _C)], idx_d, semi)

        def wait_idx(b):
            idx_s, idx_d, _, _, _, _, semi, _ = bufs[b]
            pltpu.make_async_copy(src_h.at[pl.ds(0, _C)], idx_s, semi).wait()
            pltpu.make_async_copy(dst_h.at[pl.ds(0, _C)], idx_d, semi).wait()

        def issue_gath(kk, b):
            idx_s, idx_d, trows, aldr, aler, _, _, semg = bufs[b]
            off = c * _ESC + cid_of(kk) * _C
            pltpu.async_copy(tbl_h.at[idx_s], trows, semg)
            pltpu.async_copy(ald_h.at[idx_d], aldr, semg)
            pltpu.async_copy(ale_h.at[pl.ds(off * 8, _C * 8)],
                             aler.at[pl.ds(0, _C * 8)], semg)

        def wait_gath(b):
            idx_s, idx_d, trows, aldr, aler, _, _, semg = bufs[b]
            pltpu.make_async_copy(tbl_h.at[idx_s], trows, semg).wait()
            pltpu.make_async_copy(ald_h.at[idx_d], aldr, semg).wait()
            pltpu.make_async_copy(ale_h.at[pl.ds(0, _C * 8)],
                                  aler.at[pl.ds(0, _C * 8)], semg).wait()

        def compute_scatter(b):
            _, idx_d, trows, aldr, aler, outb, _, _ = bufs[b]

            def edge(e, ecarry):
                va = trows[e, pl.ds(HID, 16)]
                vb = aldr[e, pl.ds(0, 16)]
                vc = aler[pl.ds(8 * e, 16)]
                t = (va + vb) + vc
                alpha = jnp.where(t >= 0, t, 0.2 * t)
                ex = jnp.exp(alpha)
                outb[e, pl.ds(HID, 16)] = ex
                for h in range(HEADS):
                    exh = ex.at[jnp.full((16,), h, jnp.int32)].get(
                        mode="promise_in_bounds")
                    outb[e, pl.ds(h * 16, 16)] = trows[e, pl.ds(h * 16, 16)] * exh
                return ecarry
            lax.fori_loop(0, _C, edge, 0, unroll=2)
            pltpu.sync_copy(outb, acc.at[idx_d], add=True)

        issue_idx(0, 0)
        issue_idx(1, 1)
        wait_idx(0)
        issue_gath(0, 0)

        def body(kk2, carry):
            for b in (0, 1):
                kk = 2 * kk2 + b

                @pl.when(valid(kk))
                def _():
                    wait_gath(b)

                @pl.when(valid(kk + 1))
                def _():
                    wait_idx(1 - b)
                    issue_gath(kk + 1, 1 - b)

                @pl.when(valid(kk))
                def _():
                    compute_scatter(b)

                @pl.when(valid(kk + 2))
                def _():
                    issue_idx(kk + 2, b)
            return carry
        lax.fori_loop(0, _NITER // 2, body, 0)

        plsc.subcore_barrier()
        pltpu.sync_copy(acc.at[pl.ds(zbase, rows_per_tec)],
                        out_h.at[c, pl.ds(zbase, rows_per_tec)])

    return k(src, dst, tbl, aldp, ale_flat)


_QC = 128
_QTEC = NEVAL // 32


def _eval_pass(eids, src, dst, hwp):
    mesh = plsc.VectorSubcoreMesh(core_axis_name="c", subcore_axis_name="s")

    @functools.partial(
        pl.kernel,
        out_type=jax.ShapeDtypeStruct((NEVAL, 2), jnp.float32),
        mesh=mesh,
        scratch_types=[
            pltpu.VMEM((_QC,), jnp.int32),
            pltpu.VMEM((_QC,), jnp.int32),
            pltpu.VMEM((_QC,), jnp.int32),
            pltpu.VMEM((_QC, 16), jnp.float32),
            pltpu.VMEM((_QC, 16), jnp.float32),
            pltpu.VMEM((_QC, 16), jnp.float32),
            pltpu.SemaphoreType.DMA,
            pltpu.SemaphoreType.DMA,
            pltpu.SemaphoreType.DMA,
        ],
        compiler_params=pltpu.CompilerParams(use_tc_tiling_on_sc=False),
    )
    def k(ei_h, src_h, dst_h, hw_h, out_h, idxe, idx1, idx2, r1, r2, qb,
          sem, sem2, sem3):
        c = lax.axis_index("c")
        s = lax.axis_index("s")
        base_q = (c * 16 + s) * _QTEC
        lane = lax.iota(jnp.int32, 16)

        def chunk(kk, carry):
            off = base_q + kk * _QC
            pltpu.sync_copy(ei_h.at[pl.ds(off, _QC)], idxe)
            d1 = pltpu.async_copy(src_h.at[idxe], idx1, sem)
            d2 = pltpu.async_copy(dst_h.at[idxe], idx2, sem)
            d1.wait()
            d2.wait()
            g1 = pltpu.async_copy(hw_h.at[idx1], r1, sem2)
            g2 = pltpu.async_copy(hw_h.at[idx2], r2, sem2)
            g1.wait()
            g2.wait()

            def ev(e, ecarry):
                q = (r1[e, pl.ds(0, 16)] + r2[e, pl.ds(0, 16)]) * 0.5
                e2 = jnp.exp(q + q)
                t = (e2 - 1.0) / (e2 + 1.0)
                t0 = t.at[jnp.full((16,), 0, jnp.int32)].get(
                    mode="promise_in_bounds")
                t1 = t.at[jnp.full((16,), 1, jnp.int32)].get(
                    mode="promise_in_bounds")
                lo = t0 - 0.25 * t1
                hi = t0 + 0.25 * t1
                qb[e, pl.ds(0, 16)] = jnp.where(
                    lane == 0, jnp.minimum(lo, hi), jnp.maximum(lo, hi))
                return ecarry
            lax.fori_loop(0, _QC, ev, 0, unroll=2)
            pltpu.sync_copy(qb.at[pl.ds(0, _QC), pl.ds(0, 2)],
                            out_h.at[pl.ds(off, _QC)])
            return carry
        lax.fori_loop(0, _QTEC // _QC, chunk, 0)

    return k(eids, src, dst, hwp)


def _fold(W, a):
    return (W.reshape(HID, HEADS, CH) * a[0][None]).sum(-1)


def _layer(x, src, dst, ale_l, W, As, Ad, b):
    xp = x @ W
    als = x @ As
    ald = x @ Ad
    z8 = jnp.zeros((N, 8), jnp.float32)
    tbl = jnp.concatenate([xp, als, z8], axis=1)
    aldp = jnp.concatenate([ald, z8], axis=1)
    accs = _edge_pass(src, dst, tbl, aldp, ale_l.reshape(-1))
    summed = accs[0] + accs[1]
    num = summed[:, :HID].reshape(N, HEADS, CH)
    den = summed[:, HID:HID + HEADS]
    out = num / (den[:, :, None] + 1e-16)
    return out.reshape(N, HID) + b


def kernel(edge_index, edge_type, edge_t, eids, ent_w, rel_w, tp_w, tp_b,
           g1_W, g1_We, g1_as, g1_ad, g1_ae, g1_b,
           g2_W, g2_We, g2_as, g2_ad, g2_ae, g2_b, out_w, out_b):
    src = edge_index[0]
    dst = edge_index[1]

    Ae1 = _fold(g1_We, g1_ae)
    Ae2 = _fold(g2_We, g2_ae)
    AeC = jnp.concatenate([Ae1, Ae2], axis=1)
    taec = tp_w @ AeC
    relc = rel_w @ AeC + (tp_b @ AeC)[None]

    ale1, ale2 = _ale_time(edge_t, edge_type, taec, relc)

    As1 = _fold(g1_W, g1_as)
    Ad1 = _fold(g1_W, g1_ad)
    As2 = _fold(g2_W, g2_as)
    Ad2 = _fold(g2_W, g2_ad)

    h1 = _layer(ent_w, src, dst, ale1, g1_W, As1, Ad1, g1_b)
    h2 = _layer(h1, src, dst, ale2, g2_W, As2, Ad2, g2_b)
    h = h1 + h2

    hw = h @ out_w + out_b[None]
    hwp = jnp.concatenate([hw, jnp.zeros((N, 14), jnp.float32)], axis=1)
    return _eval_pass(eids, src, dst, hwp)


# speedup vs baseline: 1.0269x; 1.0269x over previous
import functools
import math

import jax
import jax.numpy as jnp
from jax import lax
from jax.experimental import pallas as pl
from jax.experimental.pallas import tpu as pltpu
from jax.experimental.pallas import tpu_sc as plsc

N = 10000
E = 320000
HID = 128
HEADS = 8
CH = HID // HEADS
TDIM = 32
NREL = 200
NEVAL = 131072

_EBLK = 8000
_C = 64
_ACCW = 144
_ESC = E // 2
_NCH = _ESC // _C
_NITER = 158


def _ale_body(t_ref, et_ref, ta_ref, rel_ref, o1_ref, o2_ref):
    half = TDIM // 2
    idx = lax.broadcasted_iota(jnp.int32, (1, half), 1).astype(jnp.float32)
    freqs = jnp.exp(-math.log(10000.0) * idx / half)
    t0 = t_ref[:, 0:1]
    t1 = t_ref[:, 1:2]
    ang0 = t0 * freqs
    ang1 = t1 * freqs

    def _sin(x):
        x2 = x * x
        return x * (1.0 + x2 * (-1.0 / 6.0 + x2 * (1.0 / 120.0 - x2 / 5040.0)))

    def _cos(x):
        x2 = x * x
        return 1.0 + x2 * (-0.5 + x2 * (1.0 / 24.0 + x2 * (-1.0 / 720.0
                                                           + x2 / 40320.0)))

    sincos = jnp.concatenate(
        [_sin(ang0), _cos(ang0), _sin(ang1), _cos(ang1)], axis=1)
    st = jnp.dot(sincos, ta_ref[...], preferred_element_type=jnp.float32)
    et = et_ref[0, 0, :]
    onehot = (et[:, None] == lax.broadcasted_iota(jnp.int32, (1, NREL), 1)
              ).astype(jnp.float32)
    st = st + jnp.dot(onehot, rel_ref[...], preferred_element_type=jnp.float32)
    o1_ref[...] = st[:, :8]
    o2_ref[...] = st[:, 8:]


def _ale_time(edge_t, edge_type, taec, relc):
    grid = (E // _EBLK,)
    et3 = edge_type.reshape(E // _EBLK, 1, _EBLK)
    return pl.pallas_call(
        _ale_body,
        grid=grid,
        in_specs=[
            pl.BlockSpec((_EBLK, 2), lambda i: (i, 0)),
            pl.BlockSpec((1, 1, _EBLK), lambda i: (i, 0, 0)),
            pl.BlockSpec((64, 16), lambda i: (0, 0)),
            pl.BlockSpec((NREL, 16), lambda i: (0, 0)),
        ],
        out_specs=[
            pl.BlockSpec((_EBLK, 8), lambda i: (i, 0)),
            pl.BlockSpec((_EBLK, 8), lambda i: (i, 0)),
        ],
        out_shape=[
            jax.ShapeDtypeStruct((E, 8), jnp.float32),
            jax.ShapeDtypeStruct((E, 8), jnp.float32),
        ],
    )(edge_t, et3, taec, relc)


def _edge_pass(src, dst, tbl, aldp, ale_flat):
    mesh = plsc.VectorSubcoreMesh(core_axis_name="c", subcore_axis_name="s")

    vm = pltpu.VMEM
    per_buf = [
        vm((_C,), jnp.int32), vm((_C,), jnp.int32),
        vm((_C, _ACCW), jnp.float32), vm((_C, 16), jnp.float32),
        vm((_C * 8 + 8,), jnp.float32), vm((_C, _ACCW), jnp.float32),
        pltpu.SemaphoreType.DMA, pltpu.SemaphoreType.DMA,
    ]

    @functools.partial(
        pl.kernel,
        out_type=jax.ShapeDtypeStruct((2, N, _ACCW), jnp.float32),
        mesh=mesh,
        scratch_types=per_buf + per_buf + [
            pltpu.VMEM_SHARED((N, _ACCW), jnp.float32),
        ],
        compiler_params=pltpu.CompilerParams(use_tc_tiling_on_sc=False),
    )
    def k(src_h, dst_h, tbl_h, ald_h, ale_h, out_h, *scr):
        acc = scr[-1]
        bufs = (scr[0:8], scr[8:16])
        c = lax.axis_index("c")
        s = lax.axis_index("s")
        zero16 = jnp.zeros((16,), jnp.float32)
        outb0 = bufs[0][5]

        def zrow(i, carry):
            for j in range(_ACCW // 16):
                outb0[i, pl.ds(j * 16, 16)] = zero16
            return carry
        lax.fori_loop(0, _C, zrow, 0)
        rows_per_tec = N // 16
        zbase = s * rows_per_tec
        for r in range(rows_per_tec // _C):
            pltpu.sync_copy(outb0, acc.at[pl.ds(zbase + r * _C, _C)])
        zrem = rows_per_tec % _C
        pltpu.sync_copy(outb0.at[pl.ds(0, zrem)],
                        acc.at[pl.ds(zbase + rows_per_tec - zrem, zrem)])
        plsc.subcore_barrier()

        def cid_of(kk):
            return s + 16 * kk

        def valid(kk):
            return cid_of(kk) < _NCH

        def issue_idx(kk, b):
            idx_s, idx_d, _, _, _, _, semi, _ = bufs[b]
            off = c * _ESC + cid_of(kk) * _C
            pltpu.async_copy(src_h.at[pl.ds(off, _C)], idx_s, semi)
            pltpu.async_copy(dst_h.at[pl.ds(off, _C)], idx_d, semi)

        def wait_idx(b):
            idx_s, idx_d, _, _, _, _, semi, _ = bufs[b]
            pltpu.make_async_copy(src_h.at[pl.ds(0, _C)], idx_s, semi).wait()
            pltpu.make_async_copy(dst_h.at[pl.ds(0, _C)], idx_d, semi).wait()

        def issue_gath(kk, b):
            idx_s, idx_d, trows, aldr, aler, _, _, semg = bufs[b]
            off = c * _ESC + cid_of(kk) * _C
            pltpu.async_copy(tbl_h.at[idx_s], trows, semg)
            pltpu.async_copy(ald_h.at[idx_d], aldr, semg)
            pltpu.async_copy(ale_h.at[pl.ds(off * 8, _C * 8)],
                             aler.at[pl.ds(0, _C * 8)], semg)

        def wait_gath(b):
            idx_s, idx_d, trows, aldr, aler, _, _, semg = bufs[b]
            pltpu.make_async_copy(tbl_h.at[idx_s], trows, semg).wait()
            pltpu.make_async_copy(ald_h.at[idx_d], aldr, semg).wait()
            pltpu.make_async_copy(ale_h.at[pl.ds(0, _C * 8)],
                                  aler.at[pl.ds(0, _C * 8)], semg).wait()

        def compute_scatter(b):
            _, idx_d, trows, aldr, aler, outb, _, _ = bufs[b]

            def edge(e, ecarry):
                va = trows[e, pl.ds(HID, 16)]
                vb = aldr[e, pl.ds(0, 16)]
                vc = aler[pl.ds(8 * e, 16)]
                t = (va + vb) + vc
                alpha = jnp.where(t >= 0, t, 0.2 * t)
                ex = jnp.exp(alpha)
                outb[e, pl.ds(HID, 16)] = ex
                for h in range(HEADS):
                    exh = ex.at[jnp.full((16,), h, jnp.int32)].get(
                        mode="promise_in_bounds")
                    outb[e, pl.ds(h * 16, 16)] = trows[e, pl.ds(h * 16, 16)] * exh
                return ecarry
            lax.fori_loop(0, _C, edge, 0, unroll=2)
            pltpu.sync_copy(outb, acc.at[idx_d], add=True)

        issue_idx(0, 0)
        issue_idx(1, 1)
        wait_idx(0)
        issue_gath(0, 0)

        def body(kk2, carry):
            for b in (0, 1):
                kk = 2 * kk2 + b

                @pl.when(valid(kk))
                def _():
                    wait_gath(b)

                @pl.when(valid(kk + 1))
                def _():
                    wait_idx(1 - b)
                    issue_gath(kk + 1, 1 - b)

                @pl.when(valid(kk))
                def _():
                    compute_scatter(b)

                @pl.when(valid(kk + 2))
                def _():
                    issue_idx(kk + 2, b)
            return carry
        lax.fori_loop(0, _NITER // 2, body, 0)

        plsc.subcore_barrier()
        pltpu.sync_copy(acc.at[pl.ds(zbase, rows_per_tec)],
                        out_h.at[c, pl.ds(zbase, rows_per_tec)])

    return k(src, dst, tbl, aldp, ale_flat)


_QC = 128
_QTEC = NEVAL // 32


def _eval_pass(eids, src, dst, hwp):
    mesh = plsc.VectorSubcoreMesh(core_axis_name="c", subcore_axis_name="s")

    @functools.partial(
        pl.kernel,
        out_type=jax.ShapeDtypeStruct((NEVAL, 16), jnp.float32),
        mesh=mesh,
        scratch_types=[
            pltpu.VMEM((_QC,), jnp.int32),
            pltpu.VMEM((_QC,), jnp.int32),
            pltpu.VMEM((_QC,), jnp.int32),
            pltpu.VMEM((_QC, 16), jnp.float32),
            pltpu.VMEM((_QC, 16), jnp.float32),
            pltpu.VMEM((_QC, 16), jnp.float32),
            pltpu.SemaphoreType.DMA,
            pltpu.SemaphoreType.DMA,
            pltpu.SemaphoreType.DMA,
        ],
        compiler_params=pltpu.CompilerParams(use_tc_tiling_on_sc=False),
    )
    def k(ei_h, src_h, dst_h, hw_h, out_h, idxe, idx1, idx2, r1, r2, qb,
          sem, sem2, sem3):
        c = lax.axis_index("c")
        s = lax.axis_index("s")
        base_q = (c * 16 + s) * _QTEC
        lane = lax.iota(jnp.int32, 16)

        def chunk(kk, carry):
            off = base_q + kk * _QC
            pltpu.sync_copy(ei_h.at[pl.ds(off, _QC)], idxe)
            d1 = pltpu.async_copy(src_h.at[idxe], idx1, sem)
            d2 = pltpu.async_copy(dst_h.at[idxe], idx2, sem)
            d1.wait()
            d2.wait()
            g1 = pltpu.async_copy(hw_h.at[idx1], r1, sem2)
            g2 = pltpu.async_copy(hw_h.at[idx2], r2, sem2)
            g1.wait()
            g2.wait()

            def ev(e, ecarry):
                q = (r1[e, pl.ds(0, 16)] + r2[e, pl.ds(0, 16)]) * 0.5
                e2 = jnp.exp(q + q)
                t = (e2 - 1.0) / (e2 + 1.0)
                t0 = t.at[jnp.full((16,), 0, jnp.int32)].get(
                    mode="promise_in_bounds")
                t1 = t.at[jnp.full((16,), 1, jnp.int32)].get(
                    mode="promise_in_bounds")
                lo = t0 - 0.25 * t1
                hi = t0 + 0.25 * t1
                qb[e, pl.ds(0, 16)] = jnp.where(
                    lane == 0, jnp.minimum(lo, hi), jnp.maximum(lo, hi))
                return ecarry
            lax.fori_loop(0, _QC, ev, 0, unroll=2)
            pltpu.sync_copy(qb, out_h.at[pl.ds(off, _QC)])
            return carry
        lax.fori_loop(0, _QTEC // _QC, chunk, 0)

    return k(eids, src, dst, hwp)


_QBLK = 8192


def _finish_body(q_ref, o_ref):
    o_ref[...] = q_ref[:, 0:2]


def _finish(q):
    return pl.pallas_call(
        _finish_body,
        grid=(NEVAL // _QBLK,),
        in_specs=[pl.BlockSpec((_QBLK, 16), lambda i: (i, 0))],
        out_specs=pl.BlockSpec((_QBLK, 2), lambda i: (i, 0)),
        out_shape=jax.ShapeDtypeStruct((NEVAL, 2), jnp.float32),
    )(q)


def _fold(W, a):
    return (W.reshape(HID, HEADS, CH) * a[0][None]).sum(-1)


def _layer(x, src, dst, ale_l, W, As, Ad, b):
    xp = x @ W
    als = x @ As
    ald = x @ Ad
    z8 = jnp.zeros((N, 8), jnp.float32)
    tbl = jnp.concatenate([xp, als, z8], axis=1)
    aldp = jnp.concatenate([ald, z8], axis=1)
    accs = _edge_pass(src, dst, tbl, aldp, ale_l.reshape(-1))
    summed = accs[0] + accs[1]
    num = summed[:, :HID].reshape(N, HEADS, CH)
    den = summed[:, HID:HID + HEADS]
    out = num / (den[:, :, None] + 1e-16)
    return out.reshape(N, HID) + b


def kernel(edge_index, edge_type, edge_t, eids, ent_w, rel_w, tp_w, tp_b,
           g1_W, g1_We, g1_as, g1_ad, g1_ae, g1_b,
           g2_W, g2_We, g2_as, g2_ad, g2_ae, g2_b, out_w, out_b):
    src = edge_index[0]
    dst = edge_index[1]

    Ae1 = _fold(g1_We, g1_ae)
    Ae2 = _fold(g2_We, g2_ae)
    AeC = jnp.concatenate([Ae1, Ae2], axis=1)
    taec = tp_w @ AeC
    relc = rel_w @ AeC + (tp_b @ AeC)[None]

    ale1, ale2 = _ale_time(edge_t, edge_type, taec, relc)

    As1 = _fold(g1_W, g1_as)
    Ad1 = _fold(g1_W, g1_ad)
    As2 = _fold(g2_W, g2_as)
    Ad2 = _fold(g2_W, g2_ad)

    h1 = _layer(ent_w, src, dst, ale1, g1_W, As1, Ad1, g1_b)
    h2 = _layer(h1, src, dst, ale2, g2_W, As2, Ad2, g2_b)
    h = h1 + h2

    hw = h @ out_w + out_b[None]
    hwp = jnp.concatenate([hw, jnp.zeros((N, 14), jnp.float32)], axis=1)
    return _finish(_eval_pass(eids, src, dst, hwp))


# speedup vs baseline: 1.4720x; 1.4334x over previous
import functools
import math

import jax
import jax.numpy as jnp
from jax import lax
from jax.experimental import pallas as pl
from jax.experimental.pallas import tpu as pltpu
from jax.experimental.pallas import tpu_sc as plsc

N = 10000
E = 320000
HID = 128
HEADS = 8
CH = HID // HEADS
TDIM = 32
NREL = 200
NEVAL = 131072

_EBLK = 8000
_C = 64
_ACCW = 144
_ESC = E // 2
_NCH = _ESC // _C
_NITER = 158


def _ale_body(t_ref, et_ref, ta_ref, rel_ref, o1_ref, o2_ref):
    half = TDIM // 2
    idx = lax.broadcasted_iota(jnp.int32, (1, half), 1).astype(jnp.float32)
    freqs = jnp.exp(-math.log(10000.0) * idx / half)
    t0 = t_ref[:, 0:1]
    t1 = t_ref[:, 1:2]
    ang0 = t0 * freqs
    ang1 = t1 * freqs

    def _sin(x):
        x2 = x * x
        return x * (1.0 + x2 * (-1.0 / 6.0 + x2 * (1.0 / 120.0 - x2 / 5040.0)))

    def _cos(x):
        x2 = x * x
        return 1.0 + x2 * (-0.5 + x2 * (1.0 / 24.0 + x2 * (-1.0 / 720.0
                                                           + x2 / 40320.0)))

    sincos = jnp.concatenate(
        [_sin(ang0), _cos(ang0), _sin(ang1), _cos(ang1)], axis=1)
    st = jnp.dot(sincos, ta_ref[...], preferred_element_type=jnp.float32)
    et = et_ref[0, 0, :]
    onehot = (et[:, None] == lax.broadcasted_iota(jnp.int32, (1, NREL), 1)
              ).astype(jnp.float32)
    st = st + jnp.dot(onehot, rel_ref[...], preferred_element_type=jnp.float32)
    o1_ref[...] = st[:, :8]
    o2_ref[...] = st[:, 8:]


def _ale_time(edge_t, edge_type, taec, relc):
    grid = (E // _EBLK,)
    et3 = edge_type.reshape(E // _EBLK, 1, _EBLK)
    return pl.pallas_call(
        _ale_body,
        grid=grid,
        in_specs=[
            pl.BlockSpec((_EBLK, 2), lambda i: (i, 0)),
            pl.BlockSpec((1, 1, _EBLK), lambda i: (i, 0, 0)),
            pl.BlockSpec((64, 16), lambda i: (0, 0)),
            pl.BlockSpec((NREL, 16), lambda i: (0, 0)),
        ],
        out_specs=[
            pl.BlockSpec((_EBLK, 8), lambda i: (i, 0)),
            pl.BlockSpec((_EBLK, 8), lambda i: (i, 0)),
        ],
        out_shape=[
            jax.ShapeDtypeStruct((E, 8), jnp.float32),
            jax.ShapeDtypeStruct((E, 8), jnp.float32),
        ],
    )(edge_t, et3, taec, relc)


def _edge_pass(src, dst, tbl, aldp, ale_flat):
    mesh = plsc.VectorSubcoreMesh(core_axis_name="c", subcore_axis_name="s")

    vm = pltpu.VMEM
    per_buf = [
        vm((_C,), jnp.int32), vm((_C,), jnp.int32),
        vm((_C, _ACCW), jnp.float32), vm((_C, 16), jnp.float32),
        vm((_C * 8 + 8,), jnp.float32), vm((_C, _ACCW), jnp.float32),
        pltpu.SemaphoreType.DMA, pltpu.SemaphoreType.DMA,
    ]

    @functools.partial(
        pl.kernel,
        out_type=jax.ShapeDtypeStruct((2, N, _ACCW), jnp.float32),
        mesh=mesh,
        scratch_types=per_buf + per_buf + [
            pltpu.VMEM_SHARED((N, _ACCW), jnp.float32),
        ],
        compiler_params=pltpu.CompilerParams(use_tc_tiling_on_sc=False),
    )
    def k(src_h, dst_h, tbl_h, ald_h, ale_h, out_h, *scr):
        acc = scr[-1]
        bufs = (scr[0:8], scr[8:16])
        c = lax.axis_index("c")
        s = lax.axis_index("s")
        zero16 = jnp.zeros((16,), jnp.float32)
        outb0 = bufs[0][5]

        def zrow(i, carry):
            for j in range(_ACCW // 16):
                outb0[i, pl.ds(j * 16, 16)] = zero16
            return carry
        lax.fori_loop(0, _C, zrow, 0)
        rows_per_tec = N // 16
        zbase = s * rows_per_tec
        for r in range(rows_per_tec // _C):
            pltpu.sync_copy(outb0, acc.at[pl.ds(zbase + r * _C, _C)])
        zrem = rows_per_tec % _C
        pltpu.sync_copy(outb0.at[pl.ds(0, zrem)],
                        acc.at[pl.ds(zbase + rows_per_tec - zrem, zrem)])
        plsc.subcore_barrier()

        def cid_of(kk):
            return s + 16 * kk

        def valid(kk):
            return cid_of(kk) < _NCH

        def issue_idx(kk, b):
            idx_s, idx_d, _, _, _, _, semi, _ = bufs[b]
            off = c * _ESC + cid_of(kk) * _C
            pltpu.async_copy(src_h.at[pl.ds(off, _C)], idx_s, semi)
            pltpu.async_copy(dst_h.at[pl.ds(off, _C)], idx_d, semi)

        def wait_idx(b):
            idx_s, idx_d, _, _, _, _, semi, _ = bufs[b]
            pltpu.make_async_copy(src_h.at[pl.ds(0, _C)], idx_s, semi).wait()
            pltpu.make_async_copy(dst_h.at[pl.ds(0, _C)], idx_d, semi).wait()

        def issue_gath(kk, b):
            idx_s, idx_d, trows, aldr, aler, _, _, semg = bufs[b]
            off = c * _ESC + cid_of(kk) * _C
            pltpu.async_copy(tbl_h.at[idx_s], trows, semg)
            pltpu.async_copy(ald_h.at[idx_d], aldr, semg)
            pltpu.async_copy(ale_h.at[pl.ds(off * 8, _C * 8)],
                             aler.at[pl.ds(0, _C * 8)], semg)

        def wait_gath(b):
            idx_s, idx_d, trows, aldr, aler, _, _, semg = bufs[b]
            pltpu.make_async_copy(tbl_h.at[idx_s], trows, semg).wait()
            pltpu.make_async_copy(ald_h.at[idx_d], aldr, semg).wait()
            pltpu.make_async_copy(ale_h.at[pl.ds(0, _C * 8)],
                                  aler.at[pl.ds(0, _C * 8)], semg).wait()

        def compute_scatter(b):
            _, idx_d, trows, aldr, aler, outb, _, _ = bufs[b]

            def edge(e, ecarry):
                va = trows[e, pl.ds(HID, 16)]
                vb = aldr[e, pl.ds(0, 16)]
                vc = aler[pl.ds(8 * e, 16)]
                t = (va + vb) + vc
                alpha = jnp.where(t >= 0, t, 0.2 * t)
                ex = jnp.exp(alpha)
                outb[e, pl.ds(HID, 16)] = ex
                for h in range(HEADS):
                    exh = ex.at[jnp.full((16,), h, jnp.int32)].get(
                        mode="promise_in_bounds")
                    outb[e, pl.ds(h * 16, 16)] = trows[e, pl.ds(h * 16, 16)] * exh
                return ecarry
            lax.fori_loop(0, _C, edge, 0)
            pltpu.sync_copy(outb, acc.at[idx_d], add=True)

        issue_idx(0, 0)
        issue_idx(1, 1)
        wait_idx(0)
        issue_gath(0, 0)

        def body(kk2, carry):
            for b in (0, 1):
                kk = 2 * kk2 + b

                @pl.when(valid(kk))
                def _():
                    wait_gath(b)

                @pl.when(valid(kk + 1))
                def _():
                    wait_idx(1 - b)
                    issue_gath(kk + 1, 1 - b)

                @pl.when(valid(kk))
                def _():
                    compute_scatter(b)

                @pl.when(valid(kk + 2))
                def _():
                    issue_idx(kk + 2, b)
            return carry
        lax.fori_loop(0, _NITER // 2, body, 0)

        plsc.subcore_barrier()
        pltpu.sync_copy(acc.at[pl.ds(zbase, rows_per_tec)],
                        out_h.at[c, pl.ds(zbase, rows_per_tec)])

    return k(src, dst, tbl, aldp, ale_flat)


_QC = 128
_QTEC = NEVAL // 32


def _eval_pass(eids, src, dst, hwp):
    mesh = plsc.VectorSubcoreMesh(core_axis_name="c", subcore_axis_name="s")

    @functools.partial(
        pl.kernel,
        out_type=jax.ShapeDtypeStruct((NEVAL, 16), jnp.float32),
        mesh=mesh,
        scratch_types=[
            pltpu.VMEM((_QC,), jnp.int32),
            pltpu.VMEM((_QC,), jnp.int32),
            pltpu.VMEM((_QC,), jnp.int32),
            pltpu.VMEM((_QC, 16), jnp.float32),
            pltpu.VMEM((_QC, 16), jnp.float32),
            pltpu.VMEM((_QC, 16), jnp.float32),
            pltpu.SemaphoreType.DMA,
            pltpu.SemaphoreType.DMA,
            pltpu.SemaphoreType.DMA,
        ],
        compiler_params=pltpu.CompilerParams(use_tc_tiling_on_sc=False),
    )
    def k(ei_h, src_h, dst_h, hw_h, out_h, idxe, idx1, idx2, r1, r2, qb,
          sem, sem2, sem3):
        c = lax.axis_index("c")
        s = lax.axis_index("s")
        base_q = (c * 16 + s) * _QTEC
        lane = lax.iota(jnp.int32, 16)

        def chunk(kk, carry):
            off = base_q + kk * _QC
            pltpu.sync_copy(ei_h.at[pl.ds(off, _QC)], idxe)
            d1 = pltpu.async_copy(src_h.at[idxe], idx1, sem)
            d2 = pltpu.async_copy(dst_h.at[idxe], idx2, sem)
            d1.wait()
            d2.wait()
            g1 = pltpu.async_copy(hw_h.at[idx1], r1, sem2)
            g2 = pltpu.async_copy(hw_h.at[idx2], r2, sem2)
            g1.wait()
            g2.wait()

            def ev(e, ecarry):
                q = (r1[e, pl.ds(0, 16)] + r2[e, pl.ds(0, 16)]) * 0.5
                e2 = jnp.exp(q + q)
                t = (e2 - 1.0) / (e2 + 1.0)
                t0 = t.at[jnp.full((16,), 0, jnp.int32)].get(
                    mode="promise_in_bounds")
                t1 = t.at[jnp.full((16,), 1, jnp.int32)].get(
                    mode="promise_in_bounds")
                lo = t0 - 0.25 * t1
                hi = t0 + 0.25 * t1
                qb[e, pl.ds(0, 16)] = jnp.where(
                    lane == 0, jnp.minimum(lo, hi), jnp.maximum(lo, hi))
                return ecarry
            lax.fori_loop(0, _QC, ev, 0)
            pltpu.sync_copy(qb, out_h.at[pl.ds(off, _QC)])
            return carry
        lax.fori_loop(0, _QTEC // _QC, chunk, 0)

    return k(eids, src, dst, hwp)


_QBLK = 8192


def _finish_body(q_ref, o_ref):
    o_ref[...] = q_ref[:, 0:2]


def _finish(q):
    return pl.pallas_call(
        _finish_body,
        grid=(NEVAL // _QBLK,),
        in_specs=[pl.BlockSpec((_QBLK, 16), lambda i: (i, 0))],
        out_specs=pl.BlockSpec((_QBLK, 2), lambda i: (i, 0)),
        out_shape=jax.ShapeDtypeStruct((NEVAL, 2), jnp.float32),
    )(q)


def _fold(W, a):
    return (W.reshape(HID, HEADS, CH) * a[0][None]).sum(-1)


def _layer(x, src, dst, ale_l, W, As, Ad, b):
    xp = x @ W
    als = x @ As
    ald = x @ Ad
    z8 = jnp.zeros((N, 8), jnp.float32)
    tbl = jnp.concatenate([xp, als, z8], axis=1)
    aldp = jnp.concatenate([ald, z8], axis=1)
    accs = _edge_pass(src, dst, tbl, aldp, ale_l.reshape(-1))
    summed = accs[0] + accs[1]
    num = summed[:, :HID].reshape(N, HEADS, CH)
    den = summed[:, HID:HID + HEADS]
    out = num / (den[:, :, None] + 1e-16)
    return out.reshape(N, HID) + b


def kernel(edge_index, edge_type, edge_t, eids, ent_w, rel_w, tp_w, tp_b,
           g1_W, g1_We, g1_as, g1_ad, g1_ae, g1_b,
           g2_W, g2_We, g2_as, g2_ad, g2_ae, g2_b, out_w, out_b):
    src = edge_index[0]
    dst = edge_index[1]

    Ae1 = _fold(g1_We, g1_ae)
    Ae2 = _fold(g2_We, g2_ae)
    AeC = jnp.concatenate([Ae1, Ae2], axis=1)
    taec = tp_w @ AeC
    relc = rel_w @ AeC + (tp_b @ AeC)[None]

    ale1, ale2 = _ale_time(edge_t, edge_type, taec, relc)

    As1 = _fold(g1_W, g1_as)
    Ad1 = _fold(g1_W, g1_ad)
    As2 = _fold(g2_W, g2_as)
    Ad2 = _fold(g2_W, g2_ad)

    h1 = _layer(ent_w, src, dst, ale1, g1_W, As1, Ad1, g1_b)
    h2 = _layer(h1, src, dst, ale2, g2_W, As2, Ad2, g2_b)
    h = h1 + h2

    hw = h @ out_w + out_b[None]
    hwp = jnp.concatenate([hw, jnp.zeros((N, 14), jnp.float32)], axis=1)
    return _finish(_eval_pass(eids, src, dst, hwp))
